# Initial kernel scaffold; baseline (speedup 1.0000x reference)
import functools

import jax
import jax.numpy as jnp
from jax import lax
from jax.experimental import pallas as pl
from jax.experimental.pallas import tpu as pltpu
from jax.experimental.pallas import tpu_sc as plsc

D = 32
LANES = 16
NC, NS = 2, 16
NW = NC * NS
SQRT_D = float(D) ** 0.5

C = 2560
SUB = 128
KB = C // SUB


def _rsqrt(t):
    i = lax.bitcast_convert_type(t, jnp.int32)
    i = jnp.int32(0x5F3759DF) - (i >> 1)
    y = lax.bitcast_convert_type(i, jnp.float32)
    for _ in range(3):
        y = y * (1.5 - 0.5 * t * y * y)
    return y


def _make_emb(n):
    bpw = n // NW
    nchunk = bpw // C
    groups = C // LANES

    mesh = plsc.VectorSubcoreMesh(core_axis_name="c", subcore_axis_name="s")

    @functools.partial(
        pl.kernel,
        out_type=jax.ShapeDtypeStruct((n, D), jnp.float32),
        mesh=mesh,
        scratch_types=[
            pltpu.VMEM((KB, SUB), jnp.int32),
            pltpu.VMEM((C, D), jnp.float32),
            pltpu.SemaphoreType.DMA,
        ],
    )
    def emb(idx_hbm, table_hbm, out_hbm, idx_v, rows_v, sem):
        wid = lax.axis_index("s") * NC + lax.axis_index("c")
        iota = lax.iota(jnp.int32, LANES)

        def chunk_body(ci, carry):
            base = wid * bpw + ci * C
            pltpu.sync_copy(idx_hbm.at[pl.ds(base // SUB, KB)], idx_v)
            descs = [
                pltpu.async_copy(
                    table_hbm.at[idx_v.at[k]],
                    rows_v.at[pl.ds(k * SUB, SUB)],
                    sem,
                )
                for k in range(KB)
            ]
            for d in descs:
                d.wait()

            def group_body(g, gcarry):
                row_idx = g * LANES + iota
                acc = jnp.zeros((LANES,), jnp.float32)
                vals = []
                for j in range(D):
                    col = (iota + j) & (D - 1)
                    v = plsc.load_gather(rows_v, [row_idx, col])
                    vals.append(v)
                    acc = acc + v * v
                scale = _rsqrt(jnp.maximum(acc, 1e-24)) * SQRT_D
                for j in range(D):
                    col = (iota + j) & (D - 1)
                    plsc.store_scatter(rows_v, [row_idx, col], vals[j] * scale)
                return gcarry

            lax.fori_loop(0, groups, group_body, 0)
            pltpu.sync_copy(rows_v, out_hbm.at[pl.ds(base, C)])
            return carry

        lax.fori_loop(0, nchunk, chunk_body, 0)

    return emb


def kernel(input_ids, table):
    b, l = input_ids.shape
    n = b * l
    idx = input_ids.reshape(n).astype(jnp.int32).reshape(n // SUB, SUB)
    out = _make_emb(n)(idx, table)
    return out.reshape(b, l, D)


# speedup vs baseline: 1.2918x; 1.2918x over previous
import functools

import jax
import jax.numpy as jnp
from jax import lax
from jax.experimental import pallas as pl
from jax.experimental.pallas import tpu as pltpu
from jax.experimental.pallas import tpu_sc as plsc

D = 32
LANES = 16
NC, NS = 2, 16
NW = NC * NS
SQRT_D = float(D) ** 0.5

C = 1024
SUB = 128
KB = C // SUB


def _rsqrt(t):
    i = lax.bitcast_convert_type(t, jnp.int32)
    i = jnp.int32(0x5F3759DF) - (i >> 1)
    y = lax.bitcast_convert_type(i, jnp.float32)
    for _ in range(3):
        y = y * (1.5 - 0.5 * t * y * y)
    return y


def _make_emb(n):
    bpw = n // NW
    nchunk = bpw // C
    groups = C // LANES

    mesh = plsc.VectorSubcoreMesh(core_axis_name="c", subcore_axis_name="s")

    @functools.partial(
        pl.kernel,
        out_type=jax.ShapeDtypeStruct((n, D), jnp.float32),
        mesh=mesh,
        compiler_params=pltpu.CompilerParams(
            needs_layout_passes=False, use_tc_tiling_on_sc=False
        ),
        scratch_types=[
            pltpu.VMEM((KB, SUB), jnp.int32),
            pltpu.VMEM((C, D), jnp.float32),
            pltpu.SemaphoreType.DMA,
        ],
    )
    def emb(idx_hbm, table_hbm, out_hbm, idx_v, rows_v, sem):
        wid = lax.axis_index("s") * NC + lax.axis_index("c")
        iota = lax.iota(jnp.int32, LANES)

        def chunk_body(ci, carry):
            base = pl.multiple_of(wid * bpw + ci * C, C)
            pltpu.sync_copy(idx_hbm.at[pl.ds(pl.multiple_of(base // SUB, KB), KB)], idx_v)
            descs = [
                pltpu.async_copy(
                    table_hbm.at[idx_v.at[k]],
                    rows_v.at[pl.ds(k * SUB, SUB)],
                    sem,
                )
                for k in range(KB)
            ]
            for d in descs:
                d.wait()

            def group_body(g, gcarry):
                row_idx = g * LANES + iota
                acc = jnp.zeros((LANES,), jnp.float32)
                vals = []
                for j in range(D):
                    col = (iota + j) & (D - 1)
                    v = plsc.load_gather(rows_v, [row_idx, col])
                    vals.append(v)
                    acc = acc + v * v
                scale = _rsqrt(jnp.maximum(acc, 1e-24)) * SQRT_D
                for j in range(D):
                    col = (iota + j) & (D - 1)
                    plsc.store_scatter(rows_v, [row_idx, col], vals[j] * scale)
                return gcarry

            lax.fori_loop(0, groups, group_body, 0)
            pltpu.sync_copy(rows_v, out_hbm.at[pl.ds(base, C)])
            return carry

        lax.fori_loop(0, nchunk, chunk_body, 0)

    return emb


def kernel(input_ids, table):
    b, l = input_ids.shape
    n = b * l
    idx = input_ids.reshape(n).astype(jnp.int32).reshape(n // SUB, SUB)
    out = _make_emb(n)(idx, table)
    return out.reshape(b, l, D)


# speedup vs baseline: 1.6379x; 1.2679x over previous
import functools

import jax
import jax.numpy as jnp
from jax import lax
from jax.experimental import pallas as pl
from jax.experimental.pallas import tpu as pltpu
from jax.experimental.pallas import tpu_sc as plsc

D = 32
LANES = 16
NC, NS = 2, 16
NW = NC * NS
SQRT_D = float(D) ** 0.5

CB = 1024
SUB = 128
KB = CB // SUB


def _rsqrt(t):
    i = lax.bitcast_convert_type(t, jnp.int32)
    i = jnp.int32(0x5F3759DF) - (i >> 1)
    y = lax.bitcast_convert_type(i, jnp.float32)
    for _ in range(3):
        y = y * (1.5 - 0.5 * t * y * y)
    return y


def _make_emb(b, l):
    ntask = (b // CB) * l
    tpw = ntask // NW
    groups = CB // LANES

    mesh = plsc.VectorSubcoreMesh(core_axis_name="c", subcore_axis_name="s")

    @functools.partial(
        pl.kernel,
        out_type=jax.ShapeDtypeStruct((l, D, b), jnp.float32),
        mesh=mesh,
        compiler_params=pltpu.CompilerParams(
            needs_layout_passes=False, use_tc_tiling_on_sc=False
        ),
        scratch_types=[
            pltpu.VMEM((CB,), jnp.int32),
            pltpu.VMEM((CB, D), jnp.float32),
            pltpu.VMEM((D, CB), jnp.float32),
            pltpu.SemaphoreType.DMA,
        ],
    )
    def emb(ids_hbm, table_hbm, out_hbm, idx_v, rows_v, outt_v, sem):
        wid = lax.axis_index("s") * NC + lax.axis_index("c")
        iota = lax.iota(jnp.int32, LANES)

        def task_body(t, carry):
            f = wid * tpw + t
            li = f // (b // CB)
            b0 = (f % (b // CB)) * CB
            pltpu.sync_copy(
                ids_hbm.at[pl.ds(pl.multiple_of(f * CB, CB), CB)], idx_v
            )
            descs = [
                pltpu.async_copy(
                    table_hbm.at[idx_v.at[pl.ds(k * SUB, SUB)]],
                    rows_v.at[pl.ds(k * SUB, SUB)],
                    sem,
                )
                for k in range(KB)
            ]
            for dsc in descs:
                dsc.wait()

            def group_body(g, gcarry):
                row_idx = g * LANES + iota
                acc = jnp.zeros((LANES,), jnp.float32)
                vals = []
                for j in range(D):
                    col = (iota + j) & (D - 1)
                    v = plsc.load_gather(rows_v, [row_idx, col])
                    vals.append(v)
                    acc = acc + v * v
                scale = _rsqrt(jnp.maximum(acc, 1e-24)) * SQRT_D
                for j in range(D):
                    col = (iota + j) & (D - 1)
                    plsc.store_scatter(outt_v, [col, row_idx], vals[j] * scale)
                return gcarry

            lax.fori_loop(0, groups, group_body, 0)
            pltpu.sync_copy(
                outt_v,
                out_hbm.at[li, :, pl.ds(pl.multiple_of(b0, CB), CB)],
            )
            return carry

        lax.fori_loop(0, tpw, task_body, 0)

    return emb


def kernel(input_ids, table):
    b, l = input_ids.shape
    ids_t = input_ids.T.astype(jnp.int32).reshape(l * b)
    outp = _make_emb(b, l)(ids_t, table)
    return outp.transpose(2, 0, 1)


# speedup vs baseline: 1.9051x; 1.1632x over previous
import functools

import jax
import jax.numpy as jnp
from jax import lax
from jax.experimental import pallas as pl
from jax.experimental.pallas import tpu as pltpu
from jax.experimental.pallas import tpu_sc as plsc

D = 32
LANES = 16
NC, NS = 2, 16
NW = NC * NS
SQRT_D = float(D) ** 0.5

CB = 512
SUB = 128
KB = CB // SUB
GROUPS = CB // LANES


def _rsqrt(t):
    i = lax.bitcast_convert_type(t, jnp.int32)
    i = jnp.int32(0x5F3759DF) - (i >> 1)
    y = lax.bitcast_convert_type(i, jnp.float32)
    for _ in range(3):
        y = y * (1.5 - 0.5 * t * y * y)
    return y


def _make_emb(b, l):
    bc = b // CB
    ntask = bc * l
    tpw = ntask // NW
    npairs = tpw // 2

    mesh = plsc.VectorSubcoreMesh(core_axis_name="c", subcore_axis_name="s")

    @functools.partial(
        pl.kernel,
        out_type=jax.ShapeDtypeStruct((l, D, b), jnp.float32),
        mesh=mesh,
        compiler_params=pltpu.CompilerParams(
            needs_layout_passes=False, use_tc_tiling_on_sc=False
        ),
        scratch_types=[
            pltpu.VMEM((tpw * CB,), jnp.int32),
            pltpu.VMEM((2, CB, D), jnp.float32),
            pltpu.VMEM((2, D, CB), jnp.float32),
            pltpu.SemaphoreType.DMA,
            pltpu.SemaphoreType.DMA,
            pltpu.SemaphoreType.DMA,
            pltpu.SemaphoreType.DMA,
            pltpu.SemaphoreType.DMA,
        ],
    )
    def emb(ids_hbm, table_hbm, out_hbm, idx_v, rows_v, outt_v, sem_i,
            sem_g0, sem_g1, sem_o0, sem_o1):
        wid = lax.axis_index("s") * NC + lax.axis_index("c")
        f0 = wid * tpw
        iota = lax.iota(jnp.int32, LANES)
        sem_g = (sem_g0, sem_g1)
        sem_o = (sem_o0, sem_o1)

        def start_gather(t, s):
            for k in range(KB):
                pltpu.async_copy(
                    table_hbm.at[idx_v.at[pl.ds(t * CB + k * SUB, SUB)]],
                    rows_v.at[s, pl.ds(k * SUB, SUB)],
                    sem_g[s],
                )

        def wait_gather(t, s):
            for k in range(KB):
                pltpu.make_async_copy(
                    table_hbm.at[idx_v.at[pl.ds(t * CB + k * SUB, SUB)]],
                    rows_v.at[s, pl.ds(k * SUB, SUB)],
                    sem_g[s],
                ).wait()

        def out_dst(t):
            f = f0 + t
            li = f // bc
            b0 = (f % bc) * CB
            return out_hbm.at[li, :, pl.ds(pl.multiple_of(b0, CB), CB)]

        def start_out(t, s):
            pltpu.async_copy(outt_v.at[s], out_dst(t), sem_o[s])

        def wait_out(t, s):
            pltpu.make_async_copy(outt_v.at[s], out_dst(t), sem_o[s]).wait()

        def compute(t, s):
            def group_body(g, gcarry):
                row_idx = g * LANES + iota
                acc = jnp.zeros((LANES,), jnp.float32)
                vals = []
                for j in range(D):
                    col = (iota + j) & (D - 1)
                    v = plsc.load_gather(rows_v.at[s], [row_idx, col])
                    vals.append(v)
                    acc = acc + v * v
                scale = _rsqrt(jnp.maximum(acc, 1e-24)) * SQRT_D
                for j in range(D):
                    col = (iota + j) & (D - 1)
                    plsc.store_scatter(
                        outt_v.at[s], [col, row_idx], vals[j] * scale
                    )
                return gcarry

            lax.fori_loop(0, GROUPS, group_body, 0)

        pltpu.async_copy(
            ids_hbm.at[pl.ds(pl.multiple_of(f0 * CB, CB), tpw * CB)],
            idx_v, sem_i,
        ).wait()
        start_gather(0, 0)

        def step(t, s):
            wait_gather(t, s)

            @pl.when(t + 1 < tpw)
            def _():
                start_gather(t + 1, 1 - s)

            @pl.when(t >= 2)
            def _():
                wait_out(t - 2, s)

            compute(t, s)
            start_out(t, s)

        def pair_body(it, carry):
            step(2 * it, 0)
            step(2 * it + 1, 1)
            return carry

        lax.fori_loop(0, npairs, pair_body, 0)
        wait_out(tpw - 2, 0)
        wait_out(tpw - 1, 1)

    return emb


def kernel(input_ids, table):
    b, l = input_ids.shape
    ids_t = input_ids.T.astype(jnp.int32).reshape(l * b)
    outp = _make_emb(b, l)(ids_t, table)
    return outp.transpose(2, 0, 1)


# speedup vs baseline: 1.9155x; 1.0055x over previous
import functools

import jax
import jax.numpy as jnp
from jax import lax
from jax.experimental import pallas as pl
from jax.experimental.pallas import tpu as pltpu
from jax.experimental.pallas import tpu_sc as plsc

D = 32
LANES = 16
NC, NS = 2, 16
NW = NC * NS
SQRT_D = float(D) ** 0.5

CB = 256
SUB = 128
KB = CB // SUB
GROUPS = CB // LANES
SLOTS = 5


def _rsqrt(t):
    i = lax.bitcast_convert_type(t, jnp.int32)
    i = jnp.int32(0x5F3759DF) - (i >> 1)
    y = lax.bitcast_convert_type(i, jnp.float32)
    for _ in range(3):
        y = y * (1.5 - 0.5 * t * y * y)
    return y


def _make_emb(b, l):
    bc = b // CB
    ntask = bc * l
    tpw = ntask // NW
    nrounds = tpw // SLOTS

    mesh = plsc.VectorSubcoreMesh(core_axis_name="c", subcore_axis_name="s")

    @functools.partial(
        pl.kernel,
        out_type=jax.ShapeDtypeStruct((l, D, b), jnp.float32),
        mesh=mesh,
        compiler_params=pltpu.CompilerParams(
            needs_layout_passes=False, use_tc_tiling_on_sc=False
        ),
        scratch_types=[
            pltpu.VMEM((tpw * CB,), jnp.int32),
            pltpu.VMEM((SLOTS, CB, D), jnp.float32),
            pltpu.VMEM((SLOTS, D, CB), jnp.float32),
            pltpu.SemaphoreType.DMA,
            [pltpu.SemaphoreType.DMA] * SLOTS,
            [pltpu.SemaphoreType.DMA] * SLOTS,
        ],
    )
    def emb(ids_hbm, table_hbm, out_hbm, idx_v, rows_v, outt_v, sem_i,
            sem_g, sem_o):
        wid = lax.axis_index("s") * NC + lax.axis_index("c")
        f0 = wid * tpw
        iota = lax.iota(jnp.int32, LANES)

        def start_gather(t, s):
            for k in range(KB):
                pltpu.async_copy(
                    table_hbm.at[idx_v.at[pl.ds(t * CB + k * SUB, SUB)]],
                    rows_v.at[s, pl.ds(k * SUB, SUB)],
                    sem_g[s],
                )

        def wait_gather(t, s):
            for k in range(KB):
                pltpu.make_async_copy(
                    table_hbm.at[idx_v.at[pl.ds(t * CB + k * SUB, SUB)]],
                    rows_v.at[s, pl.ds(k * SUB, SUB)],
                    sem_g[s],
                ).wait()

        def out_dst(t):
            f = f0 + t
            li = f // bc
            b0 = (f % bc) * CB
            return out_hbm.at[li, :, pl.ds(pl.multiple_of(b0, CB), CB)]

        def start_out(t, s):
            pltpu.async_copy(outt_v.at[s], out_dst(t), sem_o[s])

        def wait_out(t, s):
            pltpu.make_async_copy(outt_v.at[s], out_dst(t), sem_o[s]).wait()

        def compute(t, s):
            def group_body(g, gcarry):
                row_idx = g * LANES + iota
                acc = jnp.zeros((LANES,), jnp.float32)
                vals = []
                for j in range(D):
                    col = (iota + j) & (D - 1)
                    v = plsc.load_gather(rows_v.at[s], [row_idx, col])
                    vals.append(v)
                    acc = acc + v * v
                scale = _rsqrt(jnp.maximum(acc, 1e-24)) * SQRT_D
                for j in range(D):
                    col = (iota + j) & (D - 1)
                    plsc.store_scatter(
                        outt_v.at[s], [col, row_idx], vals[j] * scale
                    )
                return gcarry

            lax.fori_loop(0, GROUPS, group_body, 0)

        pltpu.async_copy(
            ids_hbm.at[pl.ds(pl.multiple_of(f0 * CB, CB), tpw * CB)],
            idx_v, sem_i,
        ).wait()
        for s in range(SLOTS):
            start_gather(s, s)

        def step(t, s):
            wait_gather(t, s)

            @pl.when(t >= SLOTS)
            def _():
                wait_out(t - SLOTS, s)

            compute(t, s)
            start_out(t, s)

            @pl.when(t + SLOTS < tpw)
            def _():
                start_gather(t + SLOTS, s)

        def round_body(it, carry):
            for s in range(SLOTS):
                step(SLOTS * it + s, s)
            return carry

        lax.fori_loop(0, nrounds, round_body, 0)
        for s in range(SLOTS):
            wait_out(tpw - SLOTS + s, s)

    return emb


def kernel(input_ids, table):
    b, l = input_ids.shape
    ids_t = input_ids.T.astype(jnp.int32).reshape(l * b)
    outp = _make_emb(b, l)(ids_t, table)
    return outp.transpose(2, 0, 1)


# speedup vs baseline: 2.2467x; 1.1729x over previous
import functools

import jax
import jax.numpy as jnp
from jax import lax
from jax.experimental import pallas as pl
from jax.experimental.pallas import tpu as pltpu
from jax.experimental.pallas import tpu_sc as plsc

D = 32
LANES = 16
NC, NS = 2, 16
NW = NC * NS
SQRT_D = float(D) ** 0.5

CB = 256
SUB = 128
KB = CB // SUB
GROUPS = CB // LANES
SLOTS = 5


def _rsqrt(t):
    i = lax.bitcast_convert_type(t, jnp.int32)
    i = jnp.int32(0x5F3759DF) - (i >> 1)
    y = lax.bitcast_convert_type(i, jnp.float32)
    for _ in range(3):
        y = y * (1.5 - 0.5 * t * y * y)
    return y


def _make_emb(b, l):
    bc = b // CB
    ntask = bc * l
    tpw = ntask // NW
    nrounds = tpw // SLOTS

    mesh = plsc.VectorSubcoreMesh(core_axis_name="c", subcore_axis_name="s")

    nbb = CB // 128

    @functools.partial(
        pl.kernel,
        out_type=jax.ShapeDtypeStruct((l, D // 8, b // 128, 8, 128), jnp.float32),
        mesh=mesh,
        compiler_params=pltpu.CompilerParams(
            needs_layout_passes=False, use_tc_tiling_on_sc=False
        ),
        scratch_types=[
            pltpu.VMEM((tpw * CB,), jnp.int32),
            pltpu.VMEM((SLOTS, CB, D), jnp.float32),
            pltpu.VMEM((SLOTS, D // 8, nbb, 8, 128), jnp.float32),
            pltpu.SemaphoreType.DMA,
            [pltpu.SemaphoreType.DMA] * SLOTS,
            [pltpu.SemaphoreType.DMA] * SLOTS,
        ],
    )
    def emb(ids_hbm, table_hbm, out_hbm, idx_v, rows_v, outt_v, sem_i,
            sem_g, sem_o):
        wid = lax.axis_index("s") * NC + lax.axis_index("c")
        f0 = wid * tpw
        iota = lax.iota(jnp.int32, LANES)

        def start_gather(t, s):
            for k in range(KB):
                pltpu.async_copy(
                    table_hbm.at[idx_v.at[pl.ds(t * CB + k * SUB, SUB)]],
                    rows_v.at[s, pl.ds(k * SUB, SUB)],
                    sem_g[s],
                )

        def wait_gather(t, s):
            for k in range(KB):
                pltpu.make_async_copy(
                    table_hbm.at[idx_v.at[pl.ds(t * CB + k * SUB, SUB)]],
                    rows_v.at[s, pl.ds(k * SUB, SUB)],
                    sem_g[s],
                ).wait()

        def out_dst(t):
            f = f0 + t
            li = f // bc
            bb0 = (f % bc) * nbb
            return out_hbm.at[li, :, pl.ds(pl.multiple_of(bb0, nbb), nbb), :, :]

        def start_out(t, s):
            pltpu.async_copy(outt_v.at[s], out_dst(t), sem_o[s])

        def wait_out(t, s):
            pltpu.make_async_copy(outt_v.at[s], out_dst(t), sem_o[s]).wait()

        def compute(t, s):
            def group_body(g, gcarry):
                row_idx = g * LANES + iota
                bbl = g // 8
                bcv = (g % 8) * LANES + iota
                acc = jnp.zeros((LANES,), jnp.float32)
                vals = []
                for j in range(D):
                    col = (iota + j) & (D - 1)
                    v = plsc.load_gather(rows_v.at[s], [row_idx, col])
                    vals.append(v)
                    acc = acc + v * v
                scale = _rsqrt(jnp.maximum(acc, 1e-24)) * SQRT_D
                for j in range(D):
                    col = (iota + j) & (D - 1)
                    plsc.store_scatter(
                        outt_v.at[s],
                        [col >> 3, jnp.full((LANES,), bbl, jnp.int32),
                         col & 7, bcv],
                        vals[j] * scale,
                    )
                return gcarry

            lax.fori_loop(0, GROUPS, group_body, 0)

        pltpu.async_copy(
            ids_hbm.at[pl.ds(pl.multiple_of(f0 * CB, CB), tpw * CB)],
            idx_v, sem_i,
        ).wait()
        for s in range(SLOTS):
            start_gather(s, s)

        def step(t, s):
            wait_gather(t, s)

            @pl.when(t >= SLOTS)
            def _():
                wait_out(t - SLOTS, s)

            compute(t, s)
            start_out(t, s)

            @pl.when(t + SLOTS < tpw)
            def _():
                start_gather(t + SLOTS, s)

        def round_body(it, carry):
            for s in range(SLOTS):
                step(SLOTS * it + s, s)
            return carry

        lax.fori_loop(0, nrounds, round_body, 0)
        for s in range(SLOTS):
            wait_out(tpw - SLOTS + s, s)

    return emb


def kernel(input_ids, table):
    b, l = input_ids.shape
    ids_t = input_ids.T.astype(jnp.int32).reshape(l * b)
    outp = _make_emb(b, l)(ids_t, table)
    return outp.transpose(2, 4, 0, 1, 3).reshape(b, l, D)
